# baseline (speedup 1.0000x reference)
import functools

import jax
import jax.numpy as jnp
from jax import lax
from jax.experimental import pallas as pl
from jax.experimental.pallas import tpu as pltpu
from jax.experimental.pallas import tpu_sc as plsc

N = 10000
D = 128
E = 320000

NC = 2
NS = 16
NW = NC * NS

CHUNK = 256
NBUF = 1
NCHUNKS = E // CHUNK
CPT_LO = NCHUNKS // NW
REM = NCHUNKS - CPT_LO * NW

NPAD = 10112
RPT = NPAD // NS

_MESH = plsc.VectorSubcoreMesh(core_axis_name="c", subcore_axis_name="s")


@functools.partial(
    pl.kernel,
    out_type=jax.ShapeDtypeStruct((NC, NPAD, D), jnp.float32),
    mesh=_MESH,
    scratch_types=[
        [pltpu.VMEM((CHUNK,), jnp.int32) for _ in range(NBUF)],
        [pltpu.VMEM((CHUNK,), jnp.int32) for _ in range(NBUF)],
        [pltpu.VMEM((CHUNK, D), jnp.float32) for _ in range(NBUF)],
        pltpu.VMEM_SHARED((NPAD, D), jnp.float32),
        [pltpu.SemaphoreType.DMA for _ in range(NBUF)],
    ],
)
def _sc_aggregate(x_hbm, edges_hbm, zeros_hbm, out_hbm,
                  src_i, dst_i, rows, acc_sh, gsems):
    c = lax.axis_index("c")
    s = lax.axis_index("s")
    row0 = s * RPT
    wid = c * NS + s
    nch = CPT_LO + jnp.where(wid < REM, 1, 0)
    base = (wid * CPT_LO + jnp.minimum(wid, REM)) * CHUNK

    pltpu.sync_copy(zeros_hbm.at[pl.ds(row0, RPT), :],
                    acc_sh.at[pl.ds(row0, RPT), :])
    plsc.subcore_barrier()

    def load_and_gather(i, b):
        off = base + i * CHUNK
        pltpu.sync_copy(edges_hbm.at[pl.ds(off, CHUNK)], src_i[b])
        pltpu.sync_copy(edges_hbm.at[pl.ds(E + off, CHUNK)], dst_i[b])
        pltpu.async_copy(x_hbm.at[src_i[b]], rows[b], gsems[b])

    def wait_and_scatter(b):
        pltpu.make_async_copy(x_hbm.at[src_i[b]], rows[b], gsems[b]).wait()
        pltpu.sync_copy(rows[b], acc_sh.at[dst_i[b]], add=True)

    for b in range(NBUF):
        load_and_gather(b, b)

    def ring_body(p, carry):
        i0 = p * NBUF
        for b in range(NBUF):
            i = i0 + b
            wait_and_scatter(b)
            nxt = i + NBUF

            @pl.when(nxt < nch)
            def _():
                load_and_gather(nxt, b)

        return carry

    lax.fori_loop(0, CPT_LO // NBUF, ring_body, 0)

    for i in range(CPT_LO - CPT_LO % NBUF, CPT_LO):
        wait_and_scatter(i % NBUF)

    @pl.when(nch > CPT_LO)
    def _():
        wait_and_scatter(CPT_LO % NBUF)

    plsc.subcore_barrier()

    pltpu.sync_copy(acc_sh.at[pl.ds(row0, RPT), :],
                    out_hbm.at[c, pl.ds(row0, RPT), :])


BM = 1000


def _merge_body(p_ref, w_ref, b_ref, o_ref):
    acc = p_ref[0] + p_ref[1]
    o_ref[...] = lax.dot_general(
        acc, w_ref[...], (((1,), (1,)), ((), ())),
        preferred_element_type=jnp.float32) + b_ref[...]


def _merge(partial, w, b2d):
    return pl.pallas_call(
        _merge_body,
        grid=(N // BM,),
        in_specs=[
            pl.BlockSpec((NC, BM, D), lambda i: (0, i, 0)),
            pl.BlockSpec((D, D), lambda i: (0, 0)),
            pl.BlockSpec((1, D), lambda i: (0, 0)),
        ],
        out_specs=pl.BlockSpec((BM, D), lambda i: (i, 0)),
        out_shape=jax.ShapeDtypeStruct((N, D), jnp.float32),
    )(partial, w, b2d)


def kernel(x, edge_index, W, b):
    zeros = jnp.zeros((NPAD, D), jnp.float32)
    partial = _sc_aggregate(x, edge_index.astype(jnp.int32).reshape(2 * E), zeros)
    return _merge(partial, W, b.reshape(1, D))


# speedup vs baseline: 1.4478x; 1.4478x over previous
import functools

import jax
import jax.numpy as jnp
from jax import lax
from jax.experimental import pallas as pl
from jax.experimental.pallas import tpu as pltpu
from jax.experimental.pallas import tpu_sc as plsc

N = 10000
D = 128
E = 320000

NC = 2
NS = 16
NW = NC * NS

CHUNK = 128
NBUF = 2
NCHUNKS = E // CHUNK
CPT_LO = NCHUNKS // NW
REM = NCHUNKS - CPT_LO * NW

NPAD = 10112
RPT = NPAD // NS

_MESH = plsc.VectorSubcoreMesh(core_axis_name="c", subcore_axis_name="s")


@functools.partial(
    pl.kernel,
    out_type=jax.ShapeDtypeStruct((NC, NPAD, D), jnp.float32),
    mesh=_MESH,
    scratch_types=[
        [pltpu.VMEM((2, CHUNK), jnp.int32) for _ in range(NBUF)],
        [pltpu.VMEM((CHUNK, D), jnp.float32) for _ in range(NBUF)],
        pltpu.VMEM_SHARED((NPAD, D), jnp.float32),
        [pltpu.SemaphoreType.DMA for _ in range(NBUF)],
    ],
)
def _sc_aggregate(x_hbm, edges_hbm, zeros_hbm, out_hbm,
                  sd_i, rows, acc_sh, gsems):
    c = lax.axis_index("c")
    s = lax.axis_index("s")
    row0 = s * RPT
    wid = c * NS + s
    nch = CPT_LO + jnp.where(wid < REM, 1, 0)
    cbase = wid * CPT_LO + jnp.minimum(wid, REM)

    pltpu.sync_copy(zeros_hbm.at[pl.ds(row0, RPT), :],
                    acc_sh.at[pl.ds(row0, RPT), :])
    plsc.subcore_barrier()

    def load_and_gather(i, b):
        pltpu.sync_copy(edges_hbm.at[cbase + i], sd_i[b])
        pltpu.async_copy(x_hbm.at[sd_i[b].at[0]], rows[b], gsems[b])

    def wait_and_scatter(b):
        pltpu.make_async_copy(x_hbm.at[sd_i[b].at[0]], rows[b],
                              gsems[b]).wait()
        pltpu.sync_copy(rows[b], acc_sh.at[sd_i[b].at[1]], add=True)

    for b in range(NBUF):
        load_and_gather(b, b)

    def ring_body(p, carry):
        i0 = p * NBUF
        for b in range(NBUF):
            i = i0 + b
            wait_and_scatter(b)
            nxt = i + NBUF

            @pl.when(nxt < nch)
            def _():
                load_and_gather(nxt, b)

        return carry

    lax.fori_loop(0, CPT_LO // NBUF, ring_body, 0)

    for i in range(CPT_LO - CPT_LO % NBUF, CPT_LO):
        wait_and_scatter(i % NBUF)

    @pl.when(nch > CPT_LO)
    def _():
        wait_and_scatter(CPT_LO % NBUF)

    plsc.subcore_barrier()

    pltpu.sync_copy(acc_sh.at[pl.ds(row0, RPT), :],
                    out_hbm.at[c, pl.ds(row0, RPT), :])


BM = 1000


def _merge_body(p_ref, w_ref, b_ref, o_ref):
    acc = p_ref[0] + p_ref[1]
    o_ref[...] = lax.dot_general(
        acc, w_ref[...], (((1,), (1,)), ((), ())),
        preferred_element_type=jnp.float32) + b_ref[...]


def _merge(partial, w, b2d):
    return pl.pallas_call(
        _merge_body,
        grid=(N // BM,),
        in_specs=[
            pl.BlockSpec((NC, BM, D), lambda i: (0, i, 0)),
            pl.BlockSpec((D, D), lambda i: (0, 0)),
            pl.BlockSpec((1, D), lambda i: (0, 0)),
        ],
        out_specs=pl.BlockSpec((BM, D), lambda i: (i, 0)),
        out_shape=jax.ShapeDtypeStruct((N, D), jnp.float32),
    )(partial, w, b2d)


def kernel(x, edge_index, W, b):
    zeros = jnp.zeros((NPAD, D), jnp.float32)
    ei = edge_index.astype(jnp.int32).reshape(2, NCHUNKS, CHUNK)
    ei = jnp.swapaxes(ei, 0, 1)
    partial = _sc_aggregate(x, ei, zeros)
    return _merge(partial, W, b.reshape(1, D))


# speedup vs baseline: 1.6157x; 1.1160x over previous
import functools

import jax
import jax.numpy as jnp
from jax import lax
from jax.experimental import pallas as pl
from jax.experimental.pallas import tpu as pltpu
from jax.experimental.pallas import tpu_sc as plsc

N = 10000
D = 128
E = 320000

NC = 2
NS = 16
NW = NC * NS

CHUNK = 128
NBUF = 2
NIDX = 4
NCHUNKS = E // CHUNK
CPT_LO = NCHUNKS // NW
REM = NCHUNKS - CPT_LO * NW

NPAD = 10112
RPT = NPAD // NS

_MESH = plsc.VectorSubcoreMesh(core_axis_name="c", subcore_axis_name="s")


@functools.partial(
    pl.kernel,
    out_type=jax.ShapeDtypeStruct((NC, NPAD, D), jnp.float32),
    mesh=_MESH,
    scratch_types=[
        [pltpu.VMEM((2, CHUNK), jnp.int32) for _ in range(NIDX)],
        [pltpu.VMEM((CHUNK, D), jnp.float32) for _ in range(NBUF)],
        pltpu.VMEM_SHARED((NPAD, D), jnp.float32),
        [pltpu.SemaphoreType.DMA for _ in range(NIDX)],
        [pltpu.SemaphoreType.DMA for _ in range(NBUF)],
    ],
)
def _sc_aggregate(x_hbm, edges_hbm, zeros_hbm, out_hbm,
                  sd_i, rows, acc_sh, isems, gsems):
    c = lax.axis_index("c")
    s = lax.axis_index("s")
    row0 = s * RPT
    wid = c * NS + s
    nch = CPT_LO + jnp.where(wid < REM, 1, 0)
    cbase = wid * CPT_LO + jnp.minimum(wid, REM)

    pltpu.sync_copy(zeros_hbm.at[pl.ds(row0, RPT), :],
                    acc_sh.at[pl.ds(row0, RPT), :])
    plsc.subcore_barrier()

    def idx_start(i, b4):
        pltpu.async_copy(edges_hbm.at[cbase + i], sd_i[b4], isems[b4])

    def idx_wait(i, b4):
        pltpu.make_async_copy(edges_hbm.at[cbase + i], sd_i[b4],
                              isems[b4]).wait()

    def gather_start(i, b4, b2):
        pltpu.async_copy(x_hbm.at[sd_i[b4].at[0]], rows[b2], gsems[b2])

    def wait_and_scatter(b4, b2):
        pltpu.make_async_copy(x_hbm.at[sd_i[b4].at[0]], rows[b2],
                              gsems[b2]).wait()
        pltpu.sync_copy(rows[b2], acc_sh.at[sd_i[b4].at[1]], add=True)

    for k in range(NIDX):
        idx_start(k, k)
    for k in range(NBUF):
        idx_wait(k, k)
        gather_start(k, k, k)

    def round_body(p, carry):
        i0 = p * NIDX
        for v in range(NIDX):
            i = i0 + v
            b2 = v % NBUF
            wait_and_scatter(v, b2)

            @pl.when(i + NIDX < nch)
            def _():
                idx_start(i + NIDX, v)

            bn = (v + NBUF) % NIDX
            idx_wait(i + NBUF, bn)
            gather_start(i + NBUF, bn, b2)
        return carry

    NROUNDS = (CPT_LO - NBUF) // NIDX
    lax.fori_loop(0, NROUNDS, round_body, 0)

    for i in range(NROUNDS * NIDX, CPT_LO):
        v = i % NIDX
        b2 = i % NBUF
        wait_and_scatter(v, b2)
        nxt = i + NBUF
        if nxt <= CPT_LO:
            bn = nxt % NIDX

            @pl.when(nxt < nch)
            def _():
                idx_wait(nxt, bn)
                gather_start(nxt, bn, b2)

    @pl.when(nch > CPT_LO)
    def _():
        wait_and_scatter(CPT_LO % NIDX, CPT_LO % NBUF)

    plsc.subcore_barrier()

    pltpu.sync_copy(acc_sh.at[pl.ds(row0, RPT), :],
                    out_hbm.at[c, pl.ds(row0, RPT), :])


BM = 1000


def _merge_body(p_ref, w_ref, b_ref, o_ref):
    acc = p_ref[0] + p_ref[1]
    o_ref[...] = lax.dot_general(
        acc, w_ref[...], (((1,), (1,)), ((), ())),
        preferred_element_type=jnp.float32) + b_ref[...]


def _merge(partial, w, b2d):
    return pl.pallas_call(
        _merge_body,
        grid=(N // BM,),
        in_specs=[
            pl.BlockSpec((NC, BM, D), lambda i: (0, i, 0)),
            pl.BlockSpec((D, D), lambda i: (0, 0)),
            pl.BlockSpec((1, D), lambda i: (0, 0)),
        ],
        out_specs=pl.BlockSpec((BM, D), lambda i: (i, 0)),
        out_shape=jax.ShapeDtypeStruct((N, D), jnp.float32),
    )(partial, w, b2d)


def kernel(x, edge_index, W, b):
    zeros = jnp.zeros((NPAD, D), jnp.float32)
    ei = edge_index.astype(jnp.int32).reshape(2, NCHUNKS, CHUNK)
    ei = jnp.swapaxes(ei, 0, 1)
    partial = _sc_aggregate(x, ei, zeros)
    return _merge(partial, W, b.reshape(1, D))


# speedup vs baseline: 1.7913x; 1.1087x over previous
import functools

import jax
import jax.numpy as jnp
from jax import lax
from jax.experimental import pallas as pl
from jax.experimental.pallas import tpu as pltpu
from jax.experimental.pallas import tpu_sc as plsc

N = 10000
D = 128
E = 320000

NC = 2
NS = 16
NW = NC * NS

CHUNK = 128
NBUF = 3
NIDX = 4
NCHUNKS = E // CHUNK
CPT_LO = NCHUNKS // NW
REM = NCHUNKS - CPT_LO * NW

NPAD = 10112
RPT = NPAD // NS

_MESH = plsc.VectorSubcoreMesh(core_axis_name="c", subcore_axis_name="s")


@functools.partial(
    pl.kernel,
    out_type=jax.ShapeDtypeStruct((NC, NPAD, D), jnp.float32),
    mesh=_MESH,
    scratch_types=[
        pltpu.VMEM((NIDX, 2, CHUNK), jnp.int32),
        pltpu.VMEM((NBUF, CHUNK, D), jnp.float32),
        pltpu.VMEM_SHARED((NPAD, D), jnp.float32),
        pltpu.SemaphoreType.DMA((NIDX,)),
        pltpu.SemaphoreType.DMA((NBUF,)),
    ],
)
def _sc_aggregate(x_hbm, edges_hbm, zeros_hbm, out_hbm,
                  sd, rows, acc_sh, isems, gsems):
    c = lax.axis_index("c")
    s = lax.axis_index("s")
    row0 = s * RPT
    wid = c * NS + s
    nch = CPT_LO + jnp.where(wid < REM, 1, 0)
    cbase = wid * CPT_LO + jnp.minimum(wid, REM)

    pltpu.sync_copy(zeros_hbm.at[pl.ds(row0, RPT), :],
                    acc_sh.at[pl.ds(row0, RPT), :])
    plsc.subcore_barrier()

    def idx_start(i):
        slot = lax.rem(i, NIDX)
        pltpu.async_copy(edges_hbm.at[cbase + i], sd.at[slot], isems.at[slot])

    def idx_wait(i):
        slot = lax.rem(i, NIDX)
        pltpu.make_async_copy(edges_hbm.at[cbase + i], sd.at[slot],
                              isems.at[slot]).wait()

    def gather_start(i):
        slot = lax.rem(i, NIDX)
        b = lax.rem(i, NBUF)
        pltpu.async_copy(x_hbm.at[sd.at[slot, 0]], rows.at[b], gsems.at[b])

    def gather_wait(i):
        slot = lax.rem(i, NIDX)
        b = lax.rem(i, NBUF)
        pltpu.make_async_copy(x_hbm.at[sd.at[slot, 0]], rows.at[b],
                              gsems.at[b]).wait()

    def scatter(i):
        slot = lax.rem(i, NIDX)
        b = lax.rem(i, NBUF)
        pltpu.sync_copy(rows.at[b], acc_sh.at[sd.at[slot, 1]], add=True)

    def prol_idx(i, carry):
        idx_start(i)
        return carry

    lax.fori_loop(0, NIDX, prol_idx, 0)

    def prol_gather(i, carry):
        idx_wait(i)
        gather_start(i)
        return carry

    lax.fori_loop(0, NBUF, prol_gather, 0)

    def visit(i, carry):
        gather_wait(i)
        scatter(i)

        @pl.when(i + NIDX < nch)
        def _():
            idx_start(i + NIDX)

        @pl.when(i + NBUF < nch)
        def _():
            idx_wait(i + NBUF)
            gather_start(i + NBUF)

        return carry

    lax.fori_loop(0, nch, visit, 0)

    plsc.subcore_barrier()

    pltpu.sync_copy(acc_sh.at[pl.ds(row0, RPT), :],
                    out_hbm.at[c, pl.ds(row0, RPT), :])


BM = 1000


def _merge_body(p_ref, w_ref, b_ref, o_ref):
    acc = p_ref[0] + p_ref[1]
    o_ref[...] = lax.dot_general(
        acc, w_ref[...], (((1,), (1,)), ((), ())),
        preferred_element_type=jnp.float32) + b_ref[...]


def _merge(partial, w, b2d):
    return pl.pallas_call(
        _merge_body,
        grid=(N // BM,),
        in_specs=[
            pl.BlockSpec((NC, BM, D), lambda i: (0, i, 0)),
            pl.BlockSpec((D, D), lambda i: (0, 0)),
            pl.BlockSpec((1, D), lambda i: (0, 0)),
        ],
        out_specs=pl.BlockSpec((BM, D), lambda i: (i, 0)),
        out_shape=jax.ShapeDtypeStruct((N, D), jnp.float32),
    )(partial, w, b2d)


def kernel(x, edge_index, W, b):
    zeros = jnp.zeros((NPAD, D), jnp.float32)
    ei = edge_index.astype(jnp.int32).reshape(2, NCHUNKS, CHUNK)
    ei = jnp.swapaxes(ei, 0, 1)
    partial = _sc_aggregate(x, ei, zeros)
    return _merge(partial, W, b.reshape(1, D))
